# baseline (speedup 1.0000x reference)
import functools

import jax
import jax.numpy as jnp
from jax import lax
from jax.experimental import pallas as pl
from jax.experimental.pallas import tpu as pltpu
from jax.experimental.pallas import tpu_sc as plsc

_K = 8192
_CODE_DIM = 32
_TOKENS = 8
_BATCH = 1024
_T = 50
_D_MODEL = 768
_FAN_IN = _TOKENS * _CODE_DIM

_NW = 32
_STREAM = 128
_J_PER_BLK = 5
_BLK_G = _J_PER_BLK * _STREAM
_CHUNKS = (10, 20, 20)


def _sc_gather(table, idx2d, tc):
    mesh = plsc.VectorSubcoreMesh(core_axis_name="c", subcore_axis_name="s")
    f_per_w = tc * _BATCH * _TOKENS // _NW
    g_per_w = f_per_w // 4
    n_blks = g_per_w // _BLK_G

    @functools.partial(
        pl.kernel,
        mesh=mesh,
        compiler_params=pltpu.CompilerParams(
            use_tc_tiling_on_sc=False, needs_layout_passes=False),
        out_type=jax.ShapeDtypeStruct((tc * 2 * _BATCH, 4 * _CODE_DIM),
                                      jnp.float32),
        scratch_types=[
            pltpu.VMEM((f_per_w,), jnp.int32),
            pltpu.VMEM((4 * _J_PER_BLK, _STREAM), jnp.int32),
            pltpu.VMEM((4, _BLK_G, _CODE_DIM), jnp.float32),
            pltpu.VMEM_SHARED((2 * _K, _CODE_DIM), jnp.float32),
            pltpu.SemaphoreType.DMA,
        ],
    )
    def k(table_hbm, idx_hbm, out_hbm, idx_v, sidx_v, rows_v, table_sp, sem):
        cid = lax.axis_index("c")
        sid = lax.axis_index("s")
        wid = sid * 2 + cid
        g_base = wid * g_per_w

        stage = (2 * _K) // 16
        pltpu.sync_copy(
            table_hbm.at[pl.ds(sid * stage, stage)],
            table_sp.at[pl.ds(sid * stage, stage)],
        )
        pltpu.sync_copy(idx_hbm.at[wid], idx_v)
        plsc.subcore_barrier()

        lanes4 = 4 * lax.iota(jnp.int32, 16)

        def body(blk, carry):
            p0 = blk * (_J_PER_BLK * 4 * _STREAM)
            for j2 in range(_J_PER_BLK):
                for q in range(4):
                    s = j2 * 4 + q
                    for c in range(_STREAM // 16):
                        off = p0 + j2 * 512 + q + 64 * c + lanes4
                        sidx_v[s, pl.ds(c * 16, 16)] = plsc.load_gather(
                            idx_v, [off])
            copies = []
            for j2 in range(_J_PER_BLK):
                for q in range(4):
                    copies.append(
                        pltpu.async_copy(
                            table_sp.at[sidx_v.at[j2 * 4 + q]],
                            rows_v.at[q, pl.ds(j2 * _STREAM, _STREAM)],
                            sem,
                        )
                    )
            for c in copies:
                c.wait()
            for q in range(4):
                pltpu.sync_copy(
                    rows_v.at[q],
                    out_hbm.at[pl.ds(g_base + blk * _BLK_G, _BLK_G),
                               pl.ds(q * _CODE_DIM, _CODE_DIM)],
                )
            return carry

        lax.fori_loop(0, n_blks, body, 0)

    return k(table, idx2d)


def _make_tc_body(first):
    def body(x_ref, w_ref, b_ref, g_ref, bt_ref, *rest):
        if first:
            out_ref, pool_ref = rest
        else:
            _zin_ref, pool_in_ref, out_ref, pool_ref = rest
        i = pl.program_id(0)
        z = x_ref[...].reshape(_BATCH, _FAN_IN).astype(jnp.bfloat16)
        y = jnp.dot(z, w_ref[...], preferred_element_type=jnp.float32)
        y = y + b_ref[...]
        mean = jnp.mean(y, axis=-1, keepdims=True)
        var = jnp.mean((y - mean) ** 2, axis=-1, keepdims=True)
        zn = (y - mean) * lax.rsqrt(var + 1e-5) * g_ref[...] + bt_ref[...]
        out_ref[...] = zn

        @pl.when(i == 0)
        def _():
            if first:
                pool_ref[...] = zn * (1.0 / _T)
            else:
                pool_ref[...] = pool_in_ref[...] + zn * (1.0 / _T)

        @pl.when(i > 0)
        def _():
            pool_ref[...] += zn * (1.0 / _T)

    return body


def _tc_chunk(emb128, W, b, gamma, beta, t0, tc, z_prev, pool_prev):
    first = z_prev is None
    in_specs = [
        pl.BlockSpec((2 * _BATCH, 4 * _CODE_DIM), lambda i: (i, 0)),
        pl.BlockSpec((_FAN_IN, _D_MODEL), lambda i: (0, 0)),
        pl.BlockSpec((1, _D_MODEL), lambda i: (0, 0)),
        pl.BlockSpec((1, _D_MODEL), lambda i: (0, 0)),
        pl.BlockSpec((1, _D_MODEL), lambda i: (0, 0)),
    ]
    args = [emb128, W.astype(jnp.bfloat16), b.reshape(1, -1),
            gamma.reshape(1, -1), beta.reshape(1, -1)]
    aliases = {}
    if not first:
        in_specs.append(pl.BlockSpec((8, _D_MODEL), lambda i: (0, 0)))
        in_specs.append(pl.BlockSpec((_BATCH, _D_MODEL), lambda i: (0, 0)))
        args += [z_prev, pool_prev]
        aliases = {5: 0, 6: 1}
    return pl.pallas_call(
        _make_tc_body(first),
        grid=(tc,),
        in_specs=in_specs,
        out_specs=[
            pl.BlockSpec((_BATCH, _D_MODEL), lambda i, t0=t0: (t0 + i, 0)),
            pl.BlockSpec((_BATCH, _D_MODEL), lambda i: (0, 0)),
        ],
        out_shape=[
            jax.ShapeDtypeStruct((_T * _BATCH, _D_MODEL), jnp.float32),
            jax.ShapeDtypeStruct((_BATCH, _D_MODEL), jnp.float32),
        ],
        input_output_aliases=aliases,
    )(*args)


def kernel(idx, codebook_B, codebook_H, W, b, gamma, beta):
    table = jnp.concatenate([codebook_B, codebook_H], axis=0)
    offs = jnp.where(jnp.arange(_TOKENS, dtype=jnp.int32) >= 4, _K, 0)
    idx_adj = idx.reshape(_BATCH, _T, _TOKENS) + offs[None, None, :]
    idxt = idx_adj.transpose(1, 0, 2)

    z_buf, pool = None, None
    t0 = 0
    for tc in _CHUNKS:
        idx2d = idxt[t0:t0 + tc].reshape(_NW, -1)
        emb128 = _sc_gather(table, idx2d, tc)
        z_buf, pool = _tc_chunk(emb128, W, b, gamma, beta, t0, tc,
                                z_buf, pool)
        t0 += tc

    z = z_buf.reshape(_T, _BATCH, _D_MODEL).transpose(1, 0, 2)
    return (z, pool)


# speedup vs baseline: 1.0068x; 1.0068x over previous
import functools

import jax
import jax.numpy as jnp
from jax import lax
from jax.experimental import pallas as pl
from jax.experimental.pallas import tpu as pltpu
from jax.experimental.pallas import tpu_sc as plsc

_K = 8192
_CODE_DIM = 32
_TOKENS = 8
_BATCH = 1024
_T = 50
_D_MODEL = 768
_FAN_IN = _TOKENS * _CODE_DIM

_NW = 32
_STREAM = 128
_J_PER_BLK = 5
_BLK_G = _J_PER_BLK * _STREAM
_CHUNKS = (10, 20, 20)


def _sc_gather(table, idx2d, tc):
    mesh = plsc.VectorSubcoreMesh(core_axis_name="c", subcore_axis_name="s")
    f_per_w = tc * _BATCH * _TOKENS // _NW
    g_per_w = f_per_w // 4
    n_blks = g_per_w // _BLK_G

    @functools.partial(
        pl.kernel,
        mesh=mesh,
        compiler_params=pltpu.CompilerParams(
            use_tc_tiling_on_sc=False, needs_layout_passes=False),
        out_type=jax.ShapeDtypeStruct((tc * 2 * _BATCH, 4 * _CODE_DIM),
                                      jnp.float32),
        scratch_types=[
            pltpu.VMEM((f_per_w,), jnp.int32),
            pltpu.VMEM((4 * _J_PER_BLK, _STREAM), jnp.int32),
            pltpu.VMEM((4, _BLK_G, _CODE_DIM), jnp.float32),
            pltpu.VMEM_SHARED((2 * _K, _CODE_DIM), jnp.float32),
            pltpu.SemaphoreType.DMA,
        ],
    )
    def k(table_hbm, idx_hbm, out_hbm, idx_v, sidx_v, rows_v, table_sp, sem):
        cid = lax.axis_index("c")
        sid = lax.axis_index("s")
        wid = sid * 2 + cid
        g_base = wid * g_per_w

        stage = (2 * _K) // 16
        pltpu.sync_copy(
            table_hbm.at[pl.ds(sid * stage, stage)],
            table_sp.at[pl.ds(sid * stage, stage)],
        )
        pltpu.sync_copy(idx_hbm.at[wid], idx_v)
        plsc.subcore_barrier()

        lanes4 = 4 * lax.iota(jnp.int32, 16)

        def body(blk, carry):
            p0 = blk * (_J_PER_BLK * 4 * _STREAM)
            for j2 in range(_J_PER_BLK):
                for q in range(4):
                    s = j2 * 4 + q
                    for c in range(_STREAM // 16):
                        off = p0 + j2 * 512 + q + 64 * c + lanes4
                        sidx_v[s, pl.ds(c * 16, 16)] = plsc.load_gather(
                            idx_v, [off])
            copies = []
            for j2 in range(_J_PER_BLK):
                for q in range(4):
                    copies.append(
                        pltpu.async_copy(
                            table_sp.at[sidx_v.at[j2 * 4 + q]],
                            rows_v.at[q, pl.ds(j2 * _STREAM, _STREAM)],
                            sem,
                        )
                    )
            for c in copies:
                c.wait()
            for q in range(4):
                pltpu.sync_copy(
                    rows_v.at[q],
                    out_hbm.at[pl.ds(g_base + blk * _BLK_G, _BLK_G),
                               pl.ds(q * _CODE_DIM, _CODE_DIM)],
                )
            return carry

        lax.fori_loop(0, n_blks, body, 0)

    return k(table, idx2d)


def _make_tc_body(first):
    def body(x_ref, w_ref, b_ref, g_ref, bt_ref, *rest):
        if first:
            out_ref, pool_ref = rest
        else:
            _zin_ref, pool_in_ref, out_ref, pool_ref = rest
        i = pl.program_id(0)
        z = x_ref[...].reshape(_BATCH, _FAN_IN)
        y = jnp.dot(z, w_ref[...], preferred_element_type=jnp.float32)
        y = y + b_ref[...]
        mean = jnp.mean(y, axis=-1, keepdims=True)
        var = jnp.mean((y - mean) ** 2, axis=-1, keepdims=True)
        zn = (y - mean) * lax.rsqrt(var + 1e-5) * g_ref[...] + bt_ref[...]
        out_ref[...] = zn

        @pl.when(i == 0)
        def _():
            if first:
                pool_ref[...] = zn * (1.0 / _T)
            else:
                pool_ref[...] = pool_in_ref[...] + zn * (1.0 / _T)

        @pl.when(i > 0)
        def _():
            pool_ref[...] += zn * (1.0 / _T)

    return body


def _tc_chunk(emb128, W, b, gamma, beta, t0, tc, z_prev, pool_prev):
    first = z_prev is None
    in_specs = [
        pl.BlockSpec((2 * _BATCH, 4 * _CODE_DIM), lambda i: (i, 0)),
        pl.BlockSpec((_FAN_IN, _D_MODEL), lambda i: (0, 0)),
        pl.BlockSpec((1, _D_MODEL), lambda i: (0, 0)),
        pl.BlockSpec((1, _D_MODEL), lambda i: (0, 0)),
        pl.BlockSpec((1, _D_MODEL), lambda i: (0, 0)),
    ]
    args = [emb128, W, b.reshape(1, -1), gamma.reshape(1, -1),
            beta.reshape(1, -1)]
    aliases = {}
    if not first:
        in_specs.append(pl.BlockSpec((8, _D_MODEL), lambda i: (0, 0)))
        in_specs.append(pl.BlockSpec((_BATCH, _D_MODEL), lambda i: (0, 0)))
        args += [z_prev, pool_prev]
        aliases = {5: 0, 6: 1}
    return pl.pallas_call(
        _make_tc_body(first),
        grid=(tc,),
        in_specs=in_specs,
        out_specs=[
            pl.BlockSpec((_BATCH, _D_MODEL), lambda i, t0=t0: (t0 + i, 0)),
            pl.BlockSpec((_BATCH, _D_MODEL), lambda i: (0, 0)),
        ],
        out_shape=[
            jax.ShapeDtypeStruct((_T * _BATCH, _D_MODEL), jnp.float32),
            jax.ShapeDtypeStruct((_BATCH, _D_MODEL), jnp.float32),
        ],
        input_output_aliases=aliases,
    )(*args)


def kernel(idx, codebook_B, codebook_H, W, b, gamma, beta):
    table = jnp.concatenate([codebook_B, codebook_H], axis=0)
    offs = jnp.where(jnp.arange(_TOKENS, dtype=jnp.int32) >= 4, _K, 0)
    idx_adj = idx.reshape(_BATCH, _T, _TOKENS) + offs[None, None, :]
    idxt = idx_adj.transpose(1, 0, 2)

    z_buf, pool = None, None
    t0 = 0
    for tc in _CHUNKS:
        idx2d = idxt[t0:t0 + tc].reshape(_NW, -1)
        emb128 = _sc_gather(table, idx2d, tc)
        z_buf, pool = _tc_chunk(emb128, W, b, gamma, beta, t0, tc,
                                z_buf, pool)
        t0 += tc

    z = z_buf.reshape(_T, _BATCH, _D_MODEL).transpose(1, 0, 2)
    return (z, pool)


# speedup vs baseline: 1.1793x; 1.1713x over previous
import functools

import jax
import jax.numpy as jnp
from jax import lax
from jax.experimental import pallas as pl
from jax.experimental.pallas import tpu as pltpu
from jax.experimental.pallas import tpu_sc as plsc

_K = 8192
_CODE_DIM = 32
_TOKENS = 8
_BATCH = 1024
_T = 50
_D_MODEL = 768
_FAN_IN = _TOKENS * _CODE_DIM

_NW = 32
_STREAM = 128
_TL_PER_BLK = 1
_CHUNKS = (10, 20, 20)
_BPW = 64


def _sc_gather(cb_B, cb_H, idx, t0, tc):
    mesh = plsc.VectorSubcoreMesh(core_axis_name="c", subcore_axis_name="s")
    th = tc // 2
    n_blks = th // _TL_PER_BLK
    cols = th * _TOKENS

    @functools.partial(
        pl.kernel,
        mesh=mesh,
        compiler_params=pltpu.CompilerParams(
            use_tc_tiling_on_sc=False, needs_layout_passes=False),
        out_type=jax.ShapeDtypeStruct((tc * 2 * _BATCH, 4 * _CODE_DIM),
                                      jnp.float32),
        scratch_types=[
            pltpu.VMEM((_BPW, cols), jnp.int32),
            pltpu.VMEM((4 * _TL_PER_BLK, _STREAM), jnp.int32),
            pltpu.VMEM((4, _TL_PER_BLK * _STREAM, _CODE_DIM), jnp.float32),
            pltpu.VMEM_SHARED((2 * _K, _CODE_DIM), jnp.float32),
            pltpu.SemaphoreType.DMA,
        ],
    )
    def k(cb_b_hbm, cb_h_hbm, idx_hbm, out_hbm,
          idx_v, sidx_v, rows_v, table_sp, sem):
        cid = lax.axis_index("c")
        sid = lax.axis_index("s")
        wid = sid * 2 + cid
        wb = wid % 16
        wt = wid // 16

        stage = _K // 16
        pltpu.sync_copy(
            cb_b_hbm.at[pl.ds(sid * stage, stage)],
            table_sp.at[pl.ds(sid * stage, stage)],
        )
        pltpu.sync_copy(
            cb_h_hbm.at[pl.ds(sid * stage, stage)],
            table_sp.at[pl.ds(_K + sid * stage, stage)],
        )
        pltpu.sync_copy(
            idx_hbm.at[pl.ds(wb * _BPW, _BPW),
                       pl.ds((t0 + wt * th) * _TOKENS, cols)],
            idx_v,
        )
        plsc.subcore_barrier()

        lane = lax.iota(jnp.int32, 16)
        rowpat = lane // 2
        colpat = (lane % 2) * 4
        kpat = (lane % 2) * _K

        def body(blk, carry):
            for tl2 in range(_TL_PER_BLK):
                for q in range(4):
                    s = tl2 * 4 + q
                    col0 = (blk * _TL_PER_BLK + tl2) * _TOKENS + q
                    for c in range(_STREAM // 16):
                        vals = plsc.load_gather(
                            idx_v, [8 * c + rowpat, col0 + colpat])
                        sidx_v[s, pl.ds(c * 16, 16)] = vals + kpat
            copies = []
            for tl2 in range(_TL_PER_BLK):
                for q in range(4):
                    copies.append(
                        pltpu.async_copy(
                            table_sp.at[sidx_v.at[tl2 * 4 + q]],
                            rows_v.at[q, pl.ds(tl2 * _STREAM, _STREAM)],
                            sem,
                        )
                    )
            for cp in copies:
                cp.wait()
            for tl2 in range(_TL_PER_BLK):
                g0 = ((wt * th + blk * _TL_PER_BLK + tl2) * 2 * _BATCH
                      + wb * _STREAM)
                for q in range(4):
                    pltpu.sync_copy(
                        rows_v.at[q, pl.ds(tl2 * _STREAM, _STREAM)],
                        out_hbm.at[pl.ds(g0, _STREAM),
                                   pl.ds(q * _CODE_DIM, _CODE_DIM)],
                    )
            return carry

        lax.fori_loop(0, n_blks, body, 0)

    return k(cb_B, cb_H, idx)


def _make_tc_body(first):
    def body(x_ref, w_ref, b_ref, g_ref, bt_ref, *rest):
        if first:
            out_ref, pool_ref = rest
        else:
            _zin_ref, pool_in_ref, out_ref, pool_ref = rest
        i = pl.program_id(0)
        z = x_ref[...].reshape(_BATCH, _FAN_IN)
        y = jnp.dot(z, w_ref[...], preferred_element_type=jnp.float32)
        y = y + b_ref[...]
        mean = jnp.mean(y, axis=-1, keepdims=True)
        var = jnp.mean((y - mean) ** 2, axis=-1, keepdims=True)
        zn = (y - mean) * lax.rsqrt(var + 1e-5) * g_ref[...] + bt_ref[...]
        out_ref[...] = zn

        @pl.when(i == 0)
        def _():
            if first:
                pool_ref[...] = zn * (1.0 / _T)
            else:
                pool_ref[...] = pool_in_ref[...] + zn * (1.0 / _T)

        @pl.when(i > 0)
        def _():
            pool_ref[...] += zn * (1.0 / _T)

    return body


def _tc_chunk(emb128, W, b, gamma, beta, t0, tc, z_prev, pool_prev):
    first = z_prev is None
    in_specs = [
        pl.BlockSpec((2 * _BATCH, 4 * _CODE_DIM), lambda i: (i, 0)),
        pl.BlockSpec((_FAN_IN, _D_MODEL), lambda i: (0, 0)),
        pl.BlockSpec((1, _D_MODEL), lambda i: (0, 0)),
        pl.BlockSpec((1, _D_MODEL), lambda i: (0, 0)),
        pl.BlockSpec((1, _D_MODEL), lambda i: (0, 0)),
    ]
    args = [emb128, W, b.reshape(1, -1), gamma.reshape(1, -1),
            beta.reshape(1, -1)]
    aliases = {}
    if not first:
        in_specs.append(pl.BlockSpec((8, _D_MODEL), lambda i: (0, 0)))
        in_specs.append(pl.BlockSpec((_BATCH, _D_MODEL), lambda i: (0, 0)))
        args += [z_prev, pool_prev]
        aliases = {5: 0, 6: 1}
    return pl.pallas_call(
        _make_tc_body(first),
        grid=(tc,),
        in_specs=in_specs,
        out_specs=[
            pl.BlockSpec((_BATCH, _D_MODEL), lambda i, t0=t0: (t0 + i, 0)),
            pl.BlockSpec((_BATCH, _D_MODEL), lambda i: (0, 0)),
        ],
        out_shape=[
            jax.ShapeDtypeStruct((_T * _BATCH, _D_MODEL), jnp.float32),
            jax.ShapeDtypeStruct((_BATCH, _D_MODEL), jnp.float32),
        ],
        input_output_aliases=aliases,
    )(*args)


def kernel(idx, codebook_B, codebook_H, W, b, gamma, beta):
    z_buf, pool = None, None
    t0 = 0
    for tc in _CHUNKS:
        emb128 = _sc_gather(codebook_B, codebook_H, idx, t0, tc)
        z_buf, pool = _tc_chunk(emb128, W, b, gamma, beta, t0, tc,
                                z_buf, pool)
        t0 += tc

    z = z_buf.reshape(_T, _BATCH, _D_MODEL).transpose(1, 0, 2)
    return (z, pool)


# speedup vs baseline: 1.2742x; 1.0805x over previous
import functools

import jax
import jax.numpy as jnp
from jax import lax
from jax.experimental import pallas as pl
from jax.experimental.pallas import tpu as pltpu
from jax.experimental.pallas import tpu_sc as plsc

_K = 8192
_CODE_DIM = 32
_TOKENS = 8
_BATCH = 1024
_T = 50
_D_MODEL = 768
_FAN_IN = _TOKENS * _CODE_DIM

_NW = 32
_STREAM = 128
_TL_PER_BLK = 1
_CHUNKS = (10, 20, 20)
_BPW = 64


def _sc_gather(cb_B, cb_H, idx, t0, tc):
    mesh = plsc.VectorSubcoreMesh(core_axis_name="c", subcore_axis_name="s")
    th = tc // 2
    n_blks = th // _TL_PER_BLK
    cols = th * _TOKENS

    @functools.partial(
        pl.kernel,
        mesh=mesh,
        compiler_params=pltpu.CompilerParams(
            use_tc_tiling_on_sc=False, needs_layout_passes=False),
        out_type=jax.ShapeDtypeStruct((tc * 2 * _BATCH, 4 * _CODE_DIM),
                                      jnp.float32),
        scratch_types=[
            pltpu.VMEM((_BPW, cols), jnp.int32),
            pltpu.VMEM((4 * _TL_PER_BLK, _STREAM), jnp.int32),
            pltpu.VMEM((4, _TL_PER_BLK * _STREAM, _CODE_DIM), jnp.float32),
            pltpu.VMEM_SHARED((2 * _K, _CODE_DIM), jnp.float32),
            pltpu.SemaphoreType.DMA,
        ],
    )
    def k(cb_b_hbm, cb_h_hbm, idx_hbm, out_hbm,
          idx_v, sidx_v, rows_v, table_sp, sem):
        cid = lax.axis_index("c")
        sid = lax.axis_index("s")
        wid = sid * 2 + cid
        wb = wid % 16
        wt = wid // 16

        stage = _K // 16
        pltpu.sync_copy(
            cb_b_hbm.at[pl.ds(sid * stage, stage)],
            table_sp.at[pl.ds(sid * stage, stage)],
        )
        pltpu.sync_copy(
            cb_h_hbm.at[pl.ds(sid * stage, stage)],
            table_sp.at[pl.ds(_K + sid * stage, stage)],
        )
        pltpu.sync_copy(
            idx_hbm.at[pl.ds(wb * _BPW, _BPW),
                       pl.ds((t0 + wt * th) * _TOKENS, cols)],
            idx_v,
        )
        plsc.subcore_barrier()

        lane = lax.iota(jnp.int32, 16)
        rowpat = lane // 2
        colpat = (lane % 2) * 4
        kpat = (lane % 2) * _K

        def body(blk, carry):
            for tl2 in range(_TL_PER_BLK):
                for q in range(4):
                    s = tl2 * 4 + q
                    col0 = (blk * _TL_PER_BLK + tl2) * _TOKENS + q
                    for c in range(_STREAM // 16):
                        vals = plsc.load_gather(
                            idx_v, [8 * c + rowpat, col0 + colpat])
                        sidx_v[s, pl.ds(c * 16, 16)] = vals + kpat
            copies = []
            for tl2 in range(_TL_PER_BLK):
                for q in range(4):
                    copies.append(
                        pltpu.async_copy(
                            table_sp.at[sidx_v.at[tl2 * 4 + q]],
                            rows_v.at[q, pl.ds(tl2 * _STREAM, _STREAM)],
                            sem,
                        )
                    )
            for cp in copies:
                cp.wait()
            for tl2 in range(_TL_PER_BLK):
                g0 = ((wt * th + blk * _TL_PER_BLK + tl2) * 2 * _BATCH
                      + wb * _STREAM)
                for q in range(4):
                    pltpu.sync_copy(
                        rows_v.at[q, pl.ds(tl2 * _STREAM, _STREAM)],
                        out_hbm.at[pl.ds(g0, _STREAM),
                                   pl.ds(q * _CODE_DIM, _CODE_DIM)],
                    )
            return carry

        lax.fori_loop(0, n_blks, body, 0)

    return k(cb_B, cb_H, idx)


def _make_tc_body(first):
    def body(x_ref, w_ref, b_ref, g_ref, bt_ref, *rest):
        if first:
            out_ref, pool_ref = rest
        else:
            _zin_ref, pool_in_ref, out_ref, pool_ref = rest
        i = pl.program_id(0)
        z = x_ref[...].reshape(2 * _BATCH, _FAN_IN)
        y = jnp.dot(z, w_ref[...], preferred_element_type=jnp.float32)
        y = y + b_ref[...]
        mean = jnp.mean(y, axis=-1, keepdims=True)
        var = jnp.mean((y - mean) ** 2, axis=-1, keepdims=True)
        zn = (y - mean) * lax.rsqrt(var + 1e-5) * g_ref[...] + bt_ref[...]
        out_ref[...] = zn
        pstep = (zn[:_BATCH, :] + zn[_BATCH:, :]) * (1.0 / _T)

        @pl.when(i == 0)
        def _():
            if first:
                pool_ref[...] = pstep
            else:
                pool_ref[...] = pool_in_ref[...] + pstep

        @pl.when(i > 0)
        def _():
            pool_ref[...] += pstep

    return body


def _tc_chunk(emb128, W, b, gamma, beta, t0, tc, z_prev, pool_prev):
    first = z_prev is None
    in_specs = [
        pl.BlockSpec((4 * _BATCH, 4 * _CODE_DIM), lambda i: (i, 0)),
        pl.BlockSpec((_FAN_IN, _D_MODEL), lambda i: (0, 0)),
        pl.BlockSpec((1, _D_MODEL), lambda i: (0, 0)),
        pl.BlockSpec((1, _D_MODEL), lambda i: (0, 0)),
        pl.BlockSpec((1, _D_MODEL), lambda i: (0, 0)),
    ]
    args = [emb128, W, b.reshape(1, -1), gamma.reshape(1, -1),
            beta.reshape(1, -1)]
    aliases = {}
    if not first:
        in_specs.append(pl.BlockSpec((8, _D_MODEL), lambda i: (0, 0)))
        in_specs.append(pl.BlockSpec((_BATCH, _D_MODEL), lambda i: (0, 0)))
        args += [z_prev, pool_prev]
        aliases = {5: 0, 6: 1}
    return pl.pallas_call(
        _make_tc_body(first),
        grid=(tc // 2,),
        in_specs=in_specs,
        out_specs=[
            pl.BlockSpec((2 * _BATCH, _D_MODEL),
                         lambda i, t0=t0: (t0 // 2 + i, 0)),
            pl.BlockSpec((_BATCH, _D_MODEL), lambda i: (0, 0)),
        ],
        out_shape=[
            jax.ShapeDtypeStruct((_T * _BATCH, _D_MODEL), jnp.float32),
            jax.ShapeDtypeStruct((_BATCH, _D_MODEL), jnp.float32),
        ],
        input_output_aliases=aliases,
    )(*args)


def kernel(idx, codebook_B, codebook_H, W, b, gamma, beta):
    z_buf, pool = None, None
    t0 = 0
    for tc in _CHUNKS:
        emb128 = _sc_gather(codebook_B, codebook_H, idx, t0, tc)
        z_buf, pool = _tc_chunk(emb128, W, b, gamma, beta, t0, tc,
                                z_buf, pool)
        t0 += tc

    z = z_buf.reshape(_T, _BATCH, _D_MODEL).transpose(1, 0, 2)
    return (z, pool)
